# baseline (speedup 1.0000x reference)
import functools

import jax
import jax.numpy as jnp
from jax import lax
from jax.experimental import pallas as pl
from jax.experimental.pallas import tpu as pltpu
from jax.experimental.pallas import tpu_sc as plsc

_NC = 2
_NS = 16
_L = 16
_NW = _NC * _NS


@functools.partial(jax.jit, static_argnames=("n", "e", "d"))
def _midpoints(data2, eidx, *, n, e, d):
    rows_per_w = d // _NW
    e2 = e // 2
    n2 = n // 2
    mesh = plsc.VectorSubcoreMesh(core_axis_name="c", subcore_axis_name="s")

    @functools.partial(
        pl.kernel,
        out_type=jax.ShapeDtypeStruct((d, n + e), jnp.float32),
        mesh=mesh,
        scratch_types=[
            pltpu.VMEM((e,), jnp.int32),
            pltpu.VMEM((n,), jnp.float32),
            pltpu.VMEM((n,), jnp.float32),
            pltpu.VMEM((n,), jnp.float32),
            pltpu.VMEM((n,), jnp.float32),
            pltpu.VMEM((e2,), jnp.float32),
            pltpu.VMEM((e2,), jnp.float32),
            pltpu.VMEM((e2,), jnp.float32),
            pltpu.VMEM((e2,), jnp.float32),
            pltpu.VMEM((n2,), jnp.float32),
            pltpu.SemaphoreType.DMA,
            pltpu.SemaphoreType.DMA,
            pltpu.SemaphoreType.DMA,
            pltpu.SemaphoreType.DMA,
            pltpu.SemaphoreType.DMA,
            pltpu.SemaphoreType.DMA,
            pltpu.SemaphoreType.DMA,
        ],
        compiler_params=pltpu.CompilerParams(needs_layout_passes=False),
    )
    def k(data_hbm, eidx_hbm, out_hbm,
          idx_v, rowa0_v, rowb0_v, rowa1_v, rowb1_v,
          mida0_v, mida1_v, midb0_v, midb1_v, zero_v,
          sem_in0, sem_in1, sem_a0, sem_a1, sem_b0, sem_b1, sem_zero):
        wid = lax.axis_index("s") * _NC + lax.axis_index("c")
        base = wid * rows_per_w
        rows_ab = ((rowa0_v, rowb0_v), (rowa1_v, rowb1_v))
        sems_in = (sem_in0, sem_in1)
        mids_ab = ((mida0_v, mida1_v), (midb0_v, midb1_v))
        sems_ab = ((sem_a0, sem_a1), (sem_b0, sem_b1))

        pltpu.sync_copy(eidx_hbm, idx_v)

        @plsc.parallel_loop(0, n2 // _L, 1, unroll=8)
        def _(i):
            zero_v[pl.ds(i * _L, _L)] = jnp.zeros((_L,), jnp.float32)

        for sp in (0, 1):
            pltpu.async_copy(data_hbm.at[base + 2 * sp], rows_ab[sp][0],
                             sems_in[sp])
            pltpu.async_copy(data_hbm.at[base + 2 * sp + 1], rows_ab[sp][1],
                             sems_in[sp])

        @pl.loop(0, rows_per_w, step=4)
        def _(r):
            for sp in (0, 1):
                ra = base + r + 2 * sp
                rb = ra + 1
                row_a, row_b = rows_ab[sp]
                pltpu.make_async_copy(data_hbm.at[ra], row_a,
                                      sems_in[sp]).wait()
                pltpu.make_async_copy(data_hbm.at[rb], row_b,
                                      sems_in[sp]).wait()

                for h in (0, 1):
                    mid_a = mids_ab[0][h]
                    mid_b = mids_ab[1][h]

                    def drain(prev_ra=ra - 2, prev_rb=rb - 2, h=h,
                              mid_a=mid_a, mid_b=mid_b):
                        pltpu.make_async_copy(
                            mid_a,
                            out_hbm.at[prev_ra, pl.ds(n + h * e2, e2)],
                            sems_ab[0][h]).wait()
                        pltpu.make_async_copy(
                            mid_b,
                            out_hbm.at[prev_rb, pl.ds(n + h * e2, e2)],
                            sems_ab[1][h]).wait()

                    if sp == 0:
                        pl.when(r > 0)(drain)
                    else:
                        drain()

                    @plsc.parallel_loop(h * (e2 // _L), (h + 1) * (e2 // _L),
                                        1, unroll=8)
                    def _(j):
                        s = j * _L
                        so = s - h * e2
                        p = idx_v[pl.ds(s, _L)]
                        i0 = jnp.bitwise_and(p, 0xFFFF)
                        i1 = lax.shift_right_logical(p, 16)
                        a0 = plsc.load_gather(row_a, [i0])
                        a1 = plsc.load_gather(row_a, [i1])
                        b0 = plsc.load_gather(row_b, [i0])
                        b1 = plsc.load_gather(row_b, [i1])
                        mid_a[pl.ds(so, _L)] = (a0 + a1) * 0.5
                        mid_b[pl.ds(so, _L)] = (b0 + b1) * 0.5

                    pltpu.async_copy(
                        mid_a, out_hbm.at[ra, pl.ds(n + h * e2, e2)],
                        sems_ab[0][h])
                    pltpu.async_copy(
                        mid_b, out_hbm.at[rb, pl.ds(n + h * e2, e2)],
                        sems_ab[1][h])

                @pl.when(r + 2 * sp + 4 < rows_per_w)
                def _():
                    pltpu.async_copy(data_hbm.at[ra + 4], row_a,
                                     sems_in[sp])
                    pltpu.async_copy(data_hbm.at[rb + 4], row_b,
                                     sems_in[sp])

                for rr in (ra, rb):
                    for zh in (0, 1):
                        pltpu.async_copy(
                            zero_v, out_hbm.at[rr, pl.ds(zh * n2, n2)],
                            sem_zero)

        last_a = base + rows_per_w - 2
        last_b = base + rows_per_w - 1
        for h in (0, 1):
            pltpu.make_async_copy(
                mids_ab[0][h], out_hbm.at[last_a, pl.ds(n + h * e2, e2)],
                sems_ab[0][h]).wait()
            pltpu.make_async_copy(
                mids_ab[1][h], out_hbm.at[last_b, pl.ds(n + h * e2, e2)],
                sems_ab[1][h]).wait()

        @pl.loop(0, rows_per_w)
        def _(r):
            for zh in (0, 1):
                pltpu.make_async_copy(
                    zero_v, out_hbm.at[base + r, pl.ds(zh * n2, n2)],
                    sem_zero).wait()

    return k(data2, eidx)


def kernel(data, edges):
    b, c, n = data.shape
    e = edges.shape[0]
    d = b * c
    data2 = data.reshape(d, n)
    eidx = edges[:, 0] | (edges[:, 1] << 16)
    out2 = _midpoints(data2, eidx, n=n, e=e, d=d)
    return out2.reshape(b, c, n + e)
